# baseline (speedup 1.0000x reference)
import math

import jax
import jax.numpy as jnp
from jax import lax
from jax.experimental import pallas as pl
from jax.experimental.pallas import tpu as pltpu
from jax.experimental.pallas import tpu_sc as plsc

EMB = 1024
SEQ = 50
_CHUNK = 40
VOCAB_PAD = 32
SCALE = math.sqrt(EMB)

_NC = 2
_NS = 16
_NW = _NC * _NS
_LANES = 16


def _emb_body(tok_hbm, table_hbm, out_hbm, rep_hbm, idx_v, buf_v,
              gs0, gs1, ss0, ss1):
    n = out_hbm.shape[0]
    b_per_w = n // _NW
    wid = lax.axis_index("s") * _NC + lax.axis_index("c")
    base = wid * b_per_w

    tbl = buf_v.at[0, pl.ds(0, VOCAB_PAD)]
    pltpu.async_copy(table_hbm, tbl, gs0).wait()

    def srow_body(r, carry):
        def vec_body(j, carry2):
            sl = pl.ds(j * _LANES, _LANES)
            buf_v[0, r, sl] = buf_v[0, r, sl] * SCALE
            return carry2

        lax.fori_loop(0, EMB // _LANES, vec_body, 0, unroll=8)
        return carry

    lax.fori_loop(0, VOCAB_PAD, srow_body, 0, unroll=False)
    rep0 = pl.multiple_of(wid * VOCAB_PAD, 8)
    pltpu.async_copy(tbl, rep_hbm.at[pl.ds(rep0, VOCAB_PAD)], gs0).wait()

    pltpu.sync_copy(tok_hbm.at[pl.ds(base, b_per_w)], idx_v)
    woff = wid * VOCAB_PAD

    def bias_body(i, carry):
        sl = pl.ds(i * _LANES, _LANES)
        idx_v[sl] = idx_v[sl] + woff
        return carry

    lax.fori_loop(0, b_per_w // _LANES, bias_body, 0, unroll=8)

    gsems = (gs0, gs1)
    ssems = (ss0, ss1)

    def start_gather(c, b):
        off = pl.multiple_of(c * _CHUNK, 8)
        pltpu.async_copy(
            rep_hbm.at[idx_v.at[pl.ds(off, _CHUNK)]], buf_v.at[b], gsems[b]
        )

    def wait_gather(b):
        pltpu.make_async_copy(
            rep_hbm.at[idx_v.at[pl.ds(0, _CHUNK)]], buf_v.at[b], gsems[b]
        ).wait()

    def start_scatter(c, b):
        off = pl.multiple_of(base + c * _CHUNK, 8)
        pltpu.async_copy(buf_v.at[b], out_hbm.at[pl.ds(off, _CHUNK)], ssems[b])

    def wait_scatter(b):
        pltpu.make_async_copy(
            buf_v.at[b], out_hbm.at[pl.ds(0, _CHUNK)], ssems[b]
        ).wait()

    nchunk = b_per_w // _CHUNK

    start_gather(0, 0)

    wait_gather(0)
    start_scatter(0, 0)
    start_gather(1, 1)
    wait_gather(1)
    start_scatter(1, 1)
    wait_scatter(0)
    start_gather(2, 0)

    def pair_body(i, carry):
        for b in range(2):
            c = i * 2 + b
            wait_gather(b)
            start_scatter(c, b)
            wait_scatter(1 - b)
            start_gather(c + 1, 1 - b)
        return carry

    lax.fori_loop(1, nchunk // 2 - 1, pair_body, 0, unroll=False)

    c = nchunk - 2
    wait_gather(0)
    start_scatter(c, 0)
    wait_scatter(1)
    start_gather(c + 1, 1)
    wait_gather(1)
    start_scatter(c + 1, 1)
    wait_scatter(0)
    wait_scatter(1)


def kernel(tokens, table):
    batch, seq = tokens.shape
    n = batch * seq
    tok_flat = tokens.reshape(n).astype(jnp.int32)
    b_per_w = n // _NW
    vocab = table.shape[0]
    table_pad = jnp.pad(table, ((0, VOCAB_PAD - vocab), (0, 0)))

    mesh = plsc.VectorSubcoreMesh(core_axis_name="c", subcore_axis_name="s")
    call = pl.kernel(
        _emb_body,
        out_type=(
            jax.ShapeDtypeStruct((n, EMB), jnp.float32),
            jax.ShapeDtypeStruct((_NW * VOCAB_PAD, EMB), jnp.float32),
        ),
        mesh=mesh,
        scratch_types=[
            pltpu.VMEM((b_per_w,), jnp.int32),
            pltpu.VMEM((2, _CHUNK, EMB), jnp.float32),
            pltpu.SemaphoreType.DMA,
            pltpu.SemaphoreType.DMA,
            pltpu.SemaphoreType.DMA,
            pltpu.SemaphoreType.DMA,
        ],
    )
    out, _ = call(tok_flat, table_pad)
    return out.reshape(batch, seq, EMB)


# speedup vs baseline: 1.3549x; 1.3549x over previous
import math

import jax
import jax.numpy as jnp
from jax import lax
from jax.experimental import pallas as pl
from jax.experimental.pallas import tpu as pltpu
from jax.experimental.pallas import tpu_sc as plsc

EMB = 1024
SEQ = 50
SEQP = 56
VOCAB_PAD = 32
_CHUNK = 16
_NBUF = 4
SCALE = math.sqrt(EMB)

_NC = 2
_NS = 16
_NW = _NC * _NS
_LANES = 16


def _emb_body(tok_hbm, table_hbm, out_hbm, rep_hbm, idx_v, buf_v,
              gs0, gs1, gs2, gs3, ss0, ss1, ss2, ss3):
    batch = out_hbm.shape[0] // SEQP
    rows_per_w = batch // _NW
    wid = lax.axis_index("s") * _NC + lax.axis_index("c")
    row0 = wid * rows_per_w
    tbase = row0 * SEQP

    gsems = (gs0, gs1, gs2, gs3)
    ssems = (ss0, ss1, ss2, ss3)

    for h in range(2):
        pltpu.async_copy(
            table_hbm.at[pl.ds(h * _CHUNK, _CHUNK)], buf_v.at[h], gsems[h]
        )
    for h in range(2):
        pltpu.make_async_copy(
            table_hbm.at[pl.ds(0, _CHUNK)], buf_v.at[h], gsems[h]
        ).wait()

    def srow_body(r, carry):
        def vec_body(j, carry2):
            sl = pl.ds(j * _LANES, _LANES)
            buf_v[r // _CHUNK, r % _CHUNK, sl] = (
                buf_v[r // _CHUNK, r % _CHUNK, sl] * SCALE
            )
            return carry2

        lax.fori_loop(0, EMB // _LANES, vec_body, 0, unroll=8)
        return carry

    for r in range(VOCAB_PAD):
        srow_body(r, 0)
    rep0 = pl.multiple_of(wid * VOCAB_PAD, 8)
    for h in range(2):
        pltpu.async_copy(
            buf_v.at[h],
            rep_hbm.at[pl.ds(rep0 + h * _CHUNK, _CHUNK)],
            gsems[h],
        )
    for h in range(2):
        pltpu.make_async_copy(
            buf_v.at[h], rep_hbm.at[pl.ds(0, _CHUNK)], gsems[h]
        ).wait()

    pltpu.sync_copy(tok_hbm.at[pl.ds(tbase, rows_per_w * SEQP)], idx_v)
    woff = wid * VOCAB_PAD

    def bias_body(i, carry):
        sl = pl.ds(i * _LANES, _LANES)
        idx_v[sl] = idx_v[sl] + woff
        return carry

    lax.fori_loop(0, rows_per_w * SEQP // _LANES, bias_body, 0, unroll=8)

    def start_gather(c, b):
        off = pl.multiple_of(c * _CHUNK, 8)
        pltpu.async_copy(
            rep_hbm.at[idx_v.at[pl.ds(off, _CHUNK)]], buf_v.at[b], gsems[b]
        )

    def wait_gather(b):
        pltpu.make_async_copy(
            rep_hbm.at[idx_v.at[pl.ds(0, _CHUNK)]], buf_v.at[b], gsems[b]
        ).wait()

    def start_scatter(c, b):
        off = pl.multiple_of(tbase + c * _CHUNK, 8)
        pltpu.async_copy(buf_v.at[b], out_hbm.at[pl.ds(off, _CHUNK)], ssems[b])

    def wait_scatter(b):
        pltpu.make_async_copy(
            buf_v.at[b], out_hbm.at[pl.ds(0, _CHUNK)], ssems[b]
        ).wait()

    nchunk = (rows_per_w * SEQP) // _CHUNK

    start_gather(0, 0)
    start_gather(1, 1)

    wait_gather(0)
    start_scatter(0, 0)
    start_gather(2, 2)
    wait_gather(1)
    start_scatter(1, 1)
    start_gather(3, 3)

    def quad_body(i, carry):
        for k in range(_NBUF):
            c = 2 + i * _NBUF + k
            b = (2 + k) % _NBUF
            wait_gather(b)
            start_scatter(c, b)
            wait_scatter(k)
            start_gather(c + 2, k)
        return carry

    lax.fori_loop(0, (nchunk - 4) // _NBUF, quad_body, 0, unroll=False)

    c = nchunk - 2
    wait_gather(c % _NBUF)
    start_scatter(c, c % _NBUF)
    wait_scatter((c - 2) % _NBUF)
    c = nchunk - 1
    wait_gather(c % _NBUF)
    start_scatter(c, c % _NBUF)
    wait_scatter((c - 2) % _NBUF)
    wait_scatter((nchunk - 2) % _NBUF)
    wait_scatter((nchunk - 1) % _NBUF)


def kernel(tokens, table):
    batch, seq = tokens.shape
    tokp = jnp.pad(tokens.astype(jnp.int32), ((0, 0), (0, SEQP - seq)))
    tok_flat = tokp.reshape(batch * SEQP)
    rows_per_w = batch // _NW
    vocab = table.shape[0]
    table_pad = jnp.pad(table, ((0, VOCAB_PAD - vocab), (0, 0)))

    mesh = plsc.VectorSubcoreMesh(core_axis_name="c", subcore_axis_name="s")
    call = pl.kernel(
        _emb_body,
        out_type=(
            jax.ShapeDtypeStruct((batch * SEQP, EMB), jnp.float32),
            jax.ShapeDtypeStruct((_NW * VOCAB_PAD, EMB), jnp.float32),
        ),
        mesh=mesh,
        scratch_types=[
            pltpu.VMEM((rows_per_w * SEQP,), jnp.int32),
            pltpu.VMEM((_NBUF, _CHUNK, EMB), jnp.float32),
            pltpu.SemaphoreType.DMA,
            pltpu.SemaphoreType.DMA,
            pltpu.SemaphoreType.DMA,
            pltpu.SemaphoreType.DMA,
            pltpu.SemaphoreType.DMA,
            pltpu.SemaphoreType.DMA,
            pltpu.SemaphoreType.DMA,
            pltpu.SemaphoreType.DMA,
        ],
    )
    out, _ = call(tok_flat, table_pad)
    return out.reshape(batch, SEQP, EMB)[:, :seq, :]
